# baseline (speedup 1.0000x reference)
import functools

import jax
import jax.numpy as jnp
import numpy as np
from jax import lax
from jax.experimental import pallas as pl
from jax.experimental.pallas import tpu as pltpu
from jax.experimental.pallas import tpu_sc as plsc

N = 10000
E = 320000
D = 128
H = 8
HID = 16
LAT = 16

AW1 = D + 16
BW = 16
AW2 = 32

NC = 2
NS = 16
NW = NC * NS
EPW = E // NW
K = 100
NCHUNK = EPW // K
NPAIR = NCHUNK // 2
RPT = N // NS
ZR = 25

_NEG = 0.2


def _lrelu(v):
    return jnp.where(v > 0, v, v * _NEG)


def _perm_orig():
    newcol = np.arange(D)
    c = newcol // 16
    l = newcol % 16
    return (l // 2) * HID + 2 * c + l % 2


def _prep1_body(x_ref, w_ref, a1_ref, b1_ref, cv_ref):
    i = pl.program_id(0)
    nsteps = pl.num_programs(0)
    hb = jnp.dot(x_ref[...], w_ref[...], preferred_element_type=jnp.float32)
    a1_ref[...] = hb[:, :AW1]
    b1_ref[...] = hb[:, AW1:]
    cur = jnp.concatenate(
        [jnp.max(hb[:, D:AW1], axis=0, keepdims=True),
         jnp.max(hb[:, AW1:], axis=0, keepdims=True),
         jnp.zeros((6, 16), jnp.float32)], axis=0)

    @pl.when(i == 0)
    def _():
        cv_ref[...] = jnp.full((8, 16), -1e30, jnp.float32)

    mx = jnp.maximum(cv_ref[...], cur)
    cv_ref[...] = mx

    @pl.when(i == nsteps - 1)
    def _():
        c = _lrelu(mx[0:1, :] + mx[1:2, :])
        cv_ref[...] = jnp.concatenate([c, mx[1:8, :]], axis=0)


def _prep1(x, wbig):
    bn = 1000
    return pl.pallas_call(
        _prep1_body,
        grid=(N // bn,),
        in_specs=[
            pl.BlockSpec((bn, D), lambda i: (i, 0)),
            pl.BlockSpec((D, AW1 + BW), lambda i: (0, 0)),
        ],
        out_specs=[
            pl.BlockSpec((bn, AW1), lambda i: (i, 0)),
            pl.BlockSpec((bn, BW), lambda i: (i, 0)),
            pl.BlockSpec((8, 16), lambda i: (0, 0)),
        ],
        out_shape=[
            jax.ShapeDtypeStruct((N, AW1), jnp.float32),
            jax.ShapeDtypeStruct((N, BW), jnp.float32),
            jax.ShapeDtypeStruct((8, 16), jnp.float32),
        ],
    )(x, wbig)


def _make_sc_edge(aw, nheads, unroll, interleaved=False):
    mesh = plsc.VectorSubcoreMesh(core_axis_name="c", subcore_axis_name="s")

    @functools.partial(
        pl.kernel, mesh=mesh,
        compiler_params=pltpu.CompilerParams(use_tc_tiling_on_sc=False),
        out_type=jax.ShapeDtypeStruct((NC, N, aw), jnp.float32),
        scratch_types=[
            pltpu.VMEM_SHARED((N, aw), jnp.float32),
            [pltpu.VMEM((K,), jnp.int32)] * 4,
            [pltpu.VMEM((K,), jnp.int32)] * 4,
            [pltpu.VMEM((K, aw), jnp.float32)] * 2,
            [pltpu.VMEM((K, BW), jnp.float32)] * 2,
            pltpu.VMEM((16,), jnp.float32),
            pltpu.VMEM((ZR, aw), jnp.float32),
            [pltpu.SemaphoreType.DMA] * 4,
            [pltpu.SemaphoreType.DMA] * 2,
            [pltpu.SemaphoreType.DMA] * 2,
            [pltpu.SemaphoreType.DMA] * 2,
        ],
    )
    def sc_edge(a_hbm, b_hbm, src_hbm, dst_hbm, cv_hbm, out_hbm,
                acc_sh, idx_s, idx_d, rows_a, rows_b, cv_v, zero_v,
                ix, ga, gb, sc):
        c = lax.axis_index("c")
        s = lax.axis_index("s")
        w = c * NS + s

        zf = jnp.zeros((16,), jnp.float32)

        def zrow(rr, _):
            for jj in range(aw // 16):
                zero_v[rr, pl.ds(jj * 16, 16)] = zf
            return _

        lax.fori_loop(0, ZR, zrow, None)

        def zcp(k, _):
            pltpu.sync_copy(zero_v, acc_sh.at[pl.ds(s * RPT + k * ZR, ZR)])
            return _

        lax.fori_loop(0, RPT // ZR, zcp, None)

        pltpu.sync_copy(cv_hbm, cv_v)
        cvec = cv_v[...]
        plsc.subcore_barrier()

        def idx_fetch(tt, q):
            pltpu.async_copy(src_hbm.at[w, tt], idx_s[q], ix[q])
            pltpu.async_copy(dst_hbm.at[w, tt], idx_d[q], ix[q])

        def wait_idx(tt, q):
            pltpu.make_async_copy(src_hbm.at[w, tt], idx_s[q], ix[q]).wait()
            pltpu.make_async_copy(dst_hbm.at[w, tt], idx_d[q], ix[q]).wait()

        def gather(p, q):
            pltpu.async_copy(a_hbm.at[idx_s[q]], rows_a[p], ga[p])
            pltpu.async_copy(b_hbm.at[idx_d[q]], rows_b[p], gb[p])

        def wait_gather(p, q):
            pltpu.make_async_copy(a_hbm.at[idx_s[q]], rows_a[p], ga[p]).wait()
            pltpu.make_async_copy(b_hbm.at[idx_d[q]], rows_b[p], gb[p]).wait()

        def scatter(p, q):
            pltpu.async_copy(rows_a[p], acc_sh.at[idx_d[q]], sc[p], add=True)

        def wait_scatter(p, q):
            pltpu.make_async_copy(rows_a[p], acc_sh.at[idx_d[q]], sc[p]).wait()

        def compute(p):
            ra = rows_a[p]
            rb = rows_b[p]

            @plsc.parallel_loop(0, K, unroll=unroll)
            def edge(r):
                tail = ra[r, pl.ds(aw - 16, 16)]
                brow = rb[r, pl.ds(0, 16)]
                ee = jnp.exp(_lrelu(tail + brow) - cvec)
                if interleaved:
                    for j in range(nheads):
                        blk = ra[r, pl.ds(j * 16, 16)]
                        ra[r, pl.ds(j * 16, 16)] = blk * ee
                else:
                    for j in range(nheads):
                        bc = jnp.broadcast_to(ee[j], (16,))
                        blk = ra[r, pl.ds(j * 16, 16)]
                        ra[r, pl.ds(j * 16, 16)] = blk * bc
                ra[r, pl.ds(aw - 16, 16)] = ee

        idx_fetch(0, 0)
        idx_fetch(1, 1)
        idx_fetch(2, 2)
        wait_idx(0, 0)
        wait_idx(1, 1)
        gather(0, 0)

        def quad(u, _):
            t0 = u * 4
            for v in range(4):
                tt = t0 + v
                p = v % 2
                wait_gather(p, v)
                compute(p)

                if v == 0:
                    @pl.when(u > 0)
                    def _():
                        wait_scatter(1 - p, (v - 1) % 4)
                else:
                    wait_scatter(1 - p, (v - 1) % 4)

                @pl.when(tt + 2 < NCHUNK)
                def _():
                    wait_idx(tt + 2, (v + 2) % 4)

                @pl.when(tt + 1 < NCHUNK)
                def _():
                    gather(1 - p, (v + 1) % 4)

                @pl.when(tt + 3 < NCHUNK)
                def _():
                    idx_fetch(tt + 3, (v + 3) % 4)

                scatter(p, v)
            return _

        lax.fori_loop(0, NCHUNK // 4, quad, None)
        wait_scatter(1, (NCHUNK - 1) % 4)
        plsc.subcore_barrier()

        def drain(k, _):
            r0 = s * RPT + k * ZR
            pltpu.sync_copy(acc_sh.at[pl.ds(r0, ZR)], out_hbm.at[c, pl.ds(r0, ZR)])
            return _

        lax.fori_loop(0, RPT // ZR, drain, None)

    return sc_edge


_sc_edge1 = _make_sc_edge(AW1, H, 10, interleaved=True)
_sc_edge2 = _make_sc_edge(AW2, 1, 16)


def _prep2_body(acca_ref, accb_ref, a1_ref, b1_ref, cv1_ref, b1v_ref,
                w2_ref, asf2_ref, adf2_ref, a2_ref, b2_ref, cv2_ref):
    i = pl.program_id(0)
    nsteps = pl.num_programs(0)
    acc = acca_ref[...] + accb_ref[...]
    msg_p = acc[:, :D]
    den16 = acc[:, D:AW1]
    h_p = a1_ref[:, :D]
    asrc16 = a1_ref[:, D:AW1]
    adst16 = b1_ref[...]
    c1 = cv1_ref[0:1, :]
    ee_s = jnp.exp(_lrelu(asrc16 + adst16) - c1)
    den16 = den16 + ee_s
    row = lax.broadcasted_iota(jnp.int32, (16, D), 0)
    colm = lax.broadcasted_iota(jnp.int32, (16, D), 1) % 16
    tmat = (row == colm).astype(jnp.float32)
    ee128 = jnp.dot(ee_s, tmat, preferred_element_type=jnp.float32)
    den_p = jnp.dot(den16, tmat, preferred_element_type=jnp.float32)
    h1p = jnp.maximum(
        (msg_p + ee128 * h_p) / (den_p + 1e-16) + b1v_ref[...], 0.0)
    h2 = jnp.dot(h1p, w2_ref[...], preferred_element_type=jnp.float32)
    lane = lax.broadcasted_iota(jnp.int32, h2.shape, 1)
    asum = jnp.sum(h2 * asf2_ref[...], axis=1, keepdims=True)
    adum = jnp.sum(h2 * adf2_ref[...], axis=1, keepdims=True)
    asrc2 = jnp.where(lane == 0, asum, 0.0)
    adst2 = jnp.where(lane == 0, adum, 0.0)
    a2_ref[...] = jnp.concatenate([h2, asrc2], axis=1)
    b2_ref[...] = adst2
    cur = jnp.concatenate(
        [jnp.max(asrc2, axis=0, keepdims=True),
         jnp.max(adst2, axis=0, keepdims=True),
         jnp.zeros((6, 16), jnp.float32)], axis=0)

    @pl.when(i == 0)
    def _():
        cv2_ref[...] = jnp.full((8, 16), -1e30, jnp.float32)

    mx = jnp.maximum(cv2_ref[...], cur)
    cv2_ref[...] = mx

    @pl.when(i == nsteps - 1)
    def _():
        c2 = _lrelu(mx[0:1, :] + mx[1:2, :])
        cv2_ref[...] = jnp.concatenate([c2, mx[1:8, :]], axis=0)


def _prep2(acca, accb, A1, B1, cv1, b1v, W2, asf2, adf2):
    bn = 1000
    return pl.pallas_call(
        _prep2_body,
        grid=(N // bn,),
        in_specs=[
            pl.BlockSpec((bn, AW1), lambda i: (i, 0)),
            pl.BlockSpec((bn, AW1), lambda i: (i, 0)),
            pl.BlockSpec((bn, AW1), lambda i: (i, 0)),
            pl.BlockSpec((bn, BW), lambda i: (i, 0)),
            pl.BlockSpec((8, 16), lambda i: (0, 0)),
            pl.BlockSpec((1, D), lambda i: (0, 0)),
            pl.BlockSpec((D, LAT), lambda i: (0, 0)),
            pl.BlockSpec((1, LAT), lambda i: (0, 0)),
            pl.BlockSpec((1, LAT), lambda i: (0, 0)),
        ],
        out_specs=[
            pl.BlockSpec((bn, AW2), lambda i: (i, 0)),
            pl.BlockSpec((bn, BW), lambda i: (i, 0)),
            pl.BlockSpec((8, 16), lambda i: (0, 0)),
        ],
        out_shape=[
            jax.ShapeDtypeStruct((N, AW2), jnp.float32),
            jax.ShapeDtypeStruct((N, BW), jnp.float32),
            jax.ShapeDtypeStruct((8, 16), jnp.float32),
        ],
    )(acca, accb, A1, B1, cv1, b1v, W2, asf2, adf2)


def _dec_body(acca_ref, accb_ref, a2_ref, b2_ref, cv2_ref, b2v_ref,
              wd_ref, bd_ref, o_ref):
    acc = acca_ref[...] + accb_ref[...]
    msg = acc[:, :LAT]
    den = acc[:, LAT:AW2]
    h2 = a2_ref[:, :LAT]
    asrc2 = a2_ref[:, LAT:AW2]
    adst2 = b2_ref[...]
    c2 = cv2_ref[0:1, :]
    ee_s = jnp.exp(_lrelu(asrc2 + adst2) - c2)
    den = den + ee_s
    lane = lax.broadcasted_iota(jnp.int32, msg.shape, 1)
    ee0 = jnp.sum(jnp.where(lane == 0, ee_s, 0.0), axis=1, keepdims=True)
    den0 = jnp.sum(jnp.where(lane == 0, den, 0.0), axis=1, keepdims=True)
    z = (msg + ee0 * h2) / (den0 + 1e-16) + b2v_ref[...]
    o_ref[...] = jnp.dot(z, wd_ref[...], preferred_element_type=jnp.float32) \
        + bd_ref[...]


def _decode(acca, accb, A2, B2, cv2, b2v, Wd, bd):
    bn = 1000
    return pl.pallas_call(
        _dec_body,
        grid=(N // bn,),
        in_specs=[
            pl.BlockSpec((bn, AW2), lambda i: (i, 0)),
            pl.BlockSpec((bn, AW2), lambda i: (i, 0)),
            pl.BlockSpec((bn, AW2), lambda i: (i, 0)),
            pl.BlockSpec((bn, BW), lambda i: (i, 0)),
            pl.BlockSpec((8, 16), lambda i: (0, 0)),
            pl.BlockSpec((1, LAT), lambda i: (0, 0)),
            pl.BlockSpec((LAT, D), lambda i: (0, 0)),
            pl.BlockSpec((1, D), lambda i: (0, 0)),
        ],
        out_specs=pl.BlockSpec((bn, D), lambda i: (i, 0)),
        out_shape=jax.ShapeDtypeStruct((N, D), jnp.float32),
    )(acca, accb, A2, B2, cv2, b2v, Wd, bd)


def kernel(x, edge_index, W1, a_src1, a_dst1, b1, W2, a_src2, a_dst2, b2,
           Wd, bd):
    orig = _perm_orig()
    pmat = jnp.zeros((D, D), jnp.float32).at[orig, np.arange(D)].set(1.0)
    m16i = (np.arange(D)[:, None] // HID ==
            np.arange(16)[None, :] // 2).astype(np.float32)
    asrc_m = a_src1.reshape(D)[:, None] * m16i
    adst_m = a_dst1.reshape(D)[:, None] * m16i
    wbig = W1 @ jnp.concatenate([pmat, asrc_m, adst_m], axis=1)
    b1p = b1[orig].reshape(1, D)
    w2p = W2[orig, :]

    A1, B1, cv1 = _prep1(x, wbig)
    srcs = edge_index[0].reshape(NW, NCHUNK, K)
    dsts = edge_index[1].reshape(NW, NCHUNK, K)
    acc1 = _sc_edge1(A1, B1, srcs, dsts, cv1[0])
    A2, B2, cv2 = _prep2(acc1[0], acc1[1], A1, B1, cv1, b1p,
                         w2p, a_src2.reshape(1, LAT), a_dst2.reshape(1, LAT))
    acc2 = _sc_edge2(A2, B2, srcs, dsts, cv2[0])
    return _decode(acc2[0], acc2[1], A2, B2, cv2, b2.reshape(1, LAT),
                   Wd, bd.reshape(1, D))


# speedup vs baseline: 1.0351x; 1.0351x over previous
import functools

import jax
import jax.numpy as jnp
import numpy as np
from jax import lax
from jax.experimental import pallas as pl
from jax.experimental.pallas import tpu as pltpu
from jax.experimental.pallas import tpu_sc as plsc

N = 10000
E = 320000
D = 128
H = 8
HID = 16
LAT = 16

AW1 = D + 16
BW = 16
AW2 = 32

NC = 2
NS = 16
NW = NC * NS
EPW = E // NW
K = 100
NCHUNK = EPW // K
NPAIR = NCHUNK // 2
RPT = N // NS
ZR = 25

_NEG = 0.2


def _lrelu(v):
    return jnp.where(v > 0, v, v * _NEG)


def _perm_orig():
    newcol = np.arange(D)
    c = newcol // 16
    l = newcol % 16
    return (l // 2) * HID + 2 * c + l % 2


def _prep1_body(x_ref, w_ref, a1_ref, b1_ref, cv_ref):
    i = pl.program_id(0)
    nsteps = pl.num_programs(0)
    hb = jnp.dot(x_ref[...], w_ref[...], preferred_element_type=jnp.float32)
    a1_ref[...] = hb[:, :AW1]
    b1_ref[...] = hb[:, AW1:]
    cur = jnp.concatenate(
        [jnp.max(hb[:, D:AW1], axis=0, keepdims=True),
         jnp.max(hb[:, AW1:], axis=0, keepdims=True),
         jnp.zeros((6, 16), jnp.float32)], axis=0)

    @pl.when(i == 0)
    def _():
        cv_ref[...] = jnp.full((8, 16), -1e30, jnp.float32)

    mx = jnp.maximum(cv_ref[...], cur)
    cv_ref[...] = mx

    @pl.when(i == nsteps - 1)
    def _():
        c = _lrelu(mx[0:1, :] + mx[1:2, :])
        cv_ref[...] = jnp.concatenate([c, mx[1:8, :]], axis=0)


def _prep1(x, wbig):
    bn = 1000
    return pl.pallas_call(
        _prep1_body,
        grid=(N // bn,),
        in_specs=[
            pl.BlockSpec((bn, D), lambda i: (i, 0)),
            pl.BlockSpec((D, AW1 + BW), lambda i: (0, 0)),
        ],
        out_specs=[
            pl.BlockSpec((bn, AW1), lambda i: (i, 0)),
            pl.BlockSpec((bn, BW), lambda i: (i, 0)),
            pl.BlockSpec((8, 16), lambda i: (0, 0)),
        ],
        out_shape=[
            jax.ShapeDtypeStruct((N, AW1), jnp.float32),
            jax.ShapeDtypeStruct((N, BW), jnp.float32),
            jax.ShapeDtypeStruct((8, 16), jnp.float32),
        ],
    )(x, wbig)


def _make_sc_edge(aw, nheads, unroll, interleaved=False):
    mesh = plsc.VectorSubcoreMesh(core_axis_name="c", subcore_axis_name="s")

    @functools.partial(
        pl.kernel, mesh=mesh,
        compiler_params=pltpu.CompilerParams(use_tc_tiling_on_sc=False),
        out_type=jax.ShapeDtypeStruct((NC, N, aw), jnp.float32),
        scratch_types=[
            pltpu.VMEM_SHARED((N, aw), jnp.float32),
            [pltpu.VMEM((K,), jnp.int32)] * 4,
            [pltpu.VMEM((K,), jnp.int32)] * 4,
            [pltpu.VMEM((K, aw), jnp.float32)] * 2,
            [pltpu.VMEM((K, BW), jnp.float32)] * 2,
            pltpu.VMEM((16,), jnp.float32),
            pltpu.VMEM((ZR, aw), jnp.float32),
            [pltpu.SemaphoreType.DMA] * 4,
            [pltpu.SemaphoreType.DMA] * 2,
            [pltpu.SemaphoreType.DMA] * 2,
            [pltpu.SemaphoreType.DMA] * 2,
        ],
    )
    def sc_edge(a_hbm, b_hbm, src_hbm, dst_hbm, cv_hbm, out_hbm,
                acc_sh, idx_s, idx_d, rows_a, rows_b, cv_v, zero_v,
                ix, ga, gb, sc):
        c = lax.axis_index("c")
        s = lax.axis_index("s")
        w = c * NS + s

        zf = jnp.zeros((16,), jnp.float32)

        def zrow(rr, _):
            for jj in range(aw // 16):
                zero_v[rr, pl.ds(jj * 16, 16)] = zf
            return _

        lax.fori_loop(0, ZR, zrow, None)

        def zcp(k, _):
            pltpu.async_copy(zero_v, acc_sh.at[pl.ds(s * RPT + k * ZR, ZR)],
                             ga[0])
            return _

        lax.fori_loop(0, RPT // ZR, zcp, None)

        def zwait(k, _):
            pltpu.make_async_copy(
                zero_v, acc_sh.at[pl.ds(s * RPT + k * ZR, ZR)], ga[0]).wait()
            return _

        lax.fori_loop(0, RPT // ZR, zwait, None)

        pltpu.sync_copy(cv_hbm, cv_v)
        cvec = cv_v[...]
        plsc.subcore_barrier()

        def idx_fetch(tt, q):
            pltpu.async_copy(src_hbm.at[w, tt], idx_s[q], ix[q])
            pltpu.async_copy(dst_hbm.at[w, tt], idx_d[q], ix[q])

        def wait_idx(tt, q):
            pltpu.make_async_copy(src_hbm.at[w, tt], idx_s[q], ix[q]).wait()
            pltpu.make_async_copy(dst_hbm.at[w, tt], idx_d[q], ix[q]).wait()

        def gather(p, q):
            pltpu.async_copy(a_hbm.at[idx_s[q]], rows_a[p], ga[p])
            pltpu.async_copy(b_hbm.at[idx_d[q]], rows_b[p], gb[p])

        def wait_gather(p, q):
            pltpu.make_async_copy(a_hbm.at[idx_s[q]], rows_a[p], ga[p]).wait()
            pltpu.make_async_copy(b_hbm.at[idx_d[q]], rows_b[p], gb[p]).wait()

        def scatter(p, q):
            pltpu.async_copy(rows_a[p], acc_sh.at[idx_d[q]], sc[p], add=True)

        def wait_scatter(p, q):
            pltpu.make_async_copy(rows_a[p], acc_sh.at[idx_d[q]], sc[p]).wait()

        def compute(p):
            ra = rows_a[p]
            rb = rows_b[p]

            @plsc.parallel_loop(0, K, unroll=unroll)
            def edge(r):
                tail = ra[r, pl.ds(aw - 16, 16)]
                brow = rb[r, pl.ds(0, 16)]
                ee = jnp.exp(_lrelu(tail + brow) - cvec)
                if interleaved:
                    for j in range(nheads):
                        blk = ra[r, pl.ds(j * 16, 16)]
                        ra[r, pl.ds(j * 16, 16)] = blk * ee
                else:
                    for j in range(nheads):
                        bc = jnp.broadcast_to(ee[j], (16,))
                        blk = ra[r, pl.ds(j * 16, 16)]
                        ra[r, pl.ds(j * 16, 16)] = blk * bc
                ra[r, pl.ds(aw - 16, 16)] = ee

        idx_fetch(0, 0)
        idx_fetch(1, 1)
        idx_fetch(2, 2)
        wait_idx(0, 0)
        wait_idx(1, 1)
        gather(0, 0)

        def quad(u, _):
            t0 = u * 4
            for v in range(4):
                tt = t0 + v
                p = v % 2
                wait_gather(p, v)
                compute(p)

                if v == 0:
                    @pl.when(u > 0)
                    def _():
                        wait_scatter(1 - p, (v - 1) % 4)
                else:
                    wait_scatter(1 - p, (v - 1) % 4)

                @pl.when(tt + 2 < NCHUNK)
                def _():
                    wait_idx(tt + 2, (v + 2) % 4)

                @pl.when(tt + 1 < NCHUNK)
                def _():
                    gather(1 - p, (v + 1) % 4)

                @pl.when(tt + 3 < NCHUNK)
                def _():
                    idx_fetch(tt + 3, (v + 3) % 4)

                scatter(p, v)
            return _

        lax.fori_loop(0, NCHUNK // 4, quad, None)
        wait_scatter(1, (NCHUNK - 1) % 4)
        plsc.subcore_barrier()

        def drain(k, _):
            r0 = s * RPT + k * ZR
            pltpu.async_copy(acc_sh.at[pl.ds(r0, ZR)],
                             out_hbm.at[c, pl.ds(r0, ZR)], ga[0])
            return _

        lax.fori_loop(0, RPT // ZR, drain, None)

        def dwait(k, _):
            r0 = s * RPT + k * ZR
            pltpu.make_async_copy(acc_sh.at[pl.ds(r0, ZR)],
                                  out_hbm.at[c, pl.ds(r0, ZR)], ga[0]).wait()
            return _

        lax.fori_loop(0, RPT // ZR, dwait, None)

    return sc_edge


_sc_edge1 = _make_sc_edge(AW1, H, 20, interleaved=True)
_sc_edge2 = _make_sc_edge(AW2, 1, 16)


def _prep2_body(acca_ref, accb_ref, a1_ref, b1_ref, cv1_ref, b1v_ref,
                w2_ref, asf2_ref, adf2_ref, a2_ref, b2_ref, cv2_ref):
    i = pl.program_id(0)
    nsteps = pl.num_programs(0)
    acc = acca_ref[...] + accb_ref[...]
    msg_p = acc[:, :D]
    den16 = acc[:, D:AW1]
    h_p = a1_ref[:, :D]
    asrc16 = a1_ref[:, D:AW1]
    adst16 = b1_ref[...]
    c1 = cv1_ref[0:1, :]
    ee_s = jnp.exp(_lrelu(asrc16 + adst16) - c1)
    den16 = den16 + ee_s
    row = lax.broadcasted_iota(jnp.int32, (16, D), 0)
    colm = lax.broadcasted_iota(jnp.int32, (16, D), 1) % 16
    tmat = (row == colm).astype(jnp.float32)
    ee128 = jnp.dot(ee_s, tmat, preferred_element_type=jnp.float32)
    den_p = jnp.dot(den16, tmat, preferred_element_type=jnp.float32)
    h1p = jnp.maximum(
        (msg_p + ee128 * h_p) / (den_p + 1e-16) + b1v_ref[...], 0.0)
    h2 = jnp.dot(h1p, w2_ref[...], preferred_element_type=jnp.float32)
    lane = lax.broadcasted_iota(jnp.int32, h2.shape, 1)
    asum = jnp.sum(h2 * asf2_ref[...], axis=1, keepdims=True)
    adum = jnp.sum(h2 * adf2_ref[...], axis=1, keepdims=True)
    asrc2 = jnp.where(lane == 0, asum, 0.0)
    adst2 = jnp.where(lane == 0, adum, 0.0)
    a2_ref[...] = jnp.concatenate([h2, asrc2], axis=1)
    b2_ref[...] = adst2
    cur = jnp.concatenate(
        [jnp.max(asrc2, axis=0, keepdims=True),
         jnp.max(adst2, axis=0, keepdims=True),
         jnp.zeros((6, 16), jnp.float32)], axis=0)

    @pl.when(i == 0)
    def _():
        cv2_ref[...] = jnp.full((8, 16), -1e30, jnp.float32)

    mx = jnp.maximum(cv2_ref[...], cur)
    cv2_ref[...] = mx

    @pl.when(i == nsteps - 1)
    def _():
        c2 = _lrelu(mx[0:1, :] + mx[1:2, :])
        cv2_ref[...] = jnp.concatenate([c2, mx[1:8, :]], axis=0)


def _prep2(acca, accb, A1, B1, cv1, b1v, W2, asf2, adf2):
    bn = 1000
    return pl.pallas_call(
        _prep2_body,
        grid=(N // bn,),
        in_specs=[
            pl.BlockSpec((bn, AW1), lambda i: (i, 0)),
            pl.BlockSpec((bn, AW1), lambda i: (i, 0)),
            pl.BlockSpec((bn, AW1), lambda i: (i, 0)),
            pl.BlockSpec((bn, BW), lambda i: (i, 0)),
            pl.BlockSpec((8, 16), lambda i: (0, 0)),
            pl.BlockSpec((1, D), lambda i: (0, 0)),
            pl.BlockSpec((D, LAT), lambda i: (0, 0)),
            pl.BlockSpec((1, LAT), lambda i: (0, 0)),
            pl.BlockSpec((1, LAT), lambda i: (0, 0)),
        ],
        out_specs=[
            pl.BlockSpec((bn, AW2), lambda i: (i, 0)),
            pl.BlockSpec((bn, BW), lambda i: (i, 0)),
            pl.BlockSpec((8, 16), lambda i: (0, 0)),
        ],
        out_shape=[
            jax.ShapeDtypeStruct((N, AW2), jnp.float32),
            jax.ShapeDtypeStruct((N, BW), jnp.float32),
            jax.ShapeDtypeStruct((8, 16), jnp.float32),
        ],
    )(acca, accb, A1, B1, cv1, b1v, W2, asf2, adf2)


def _dec_body(acca_ref, accb_ref, a2_ref, b2_ref, cv2_ref, b2v_ref,
              wd_ref, bd_ref, o_ref):
    acc = acca_ref[...] + accb_ref[...]
    msg = acc[:, :LAT]
    den = acc[:, LAT:AW2]
    h2 = a2_ref[:, :LAT]
    asrc2 = a2_ref[:, LAT:AW2]
    adst2 = b2_ref[...]
    c2 = cv2_ref[0:1, :]
    ee_s = jnp.exp(_lrelu(asrc2 + adst2) - c2)
    den = den + ee_s
    lane = lax.broadcasted_iota(jnp.int32, msg.shape, 1)
    ee0 = jnp.sum(jnp.where(lane == 0, ee_s, 0.0), axis=1, keepdims=True)
    den0 = jnp.sum(jnp.where(lane == 0, den, 0.0), axis=1, keepdims=True)
    z = (msg + ee0 * h2) / (den0 + 1e-16) + b2v_ref[...]
    o_ref[...] = jnp.dot(z, wd_ref[...], preferred_element_type=jnp.float32) \
        + bd_ref[...]


def _decode(acca, accb, A2, B2, cv2, b2v, Wd, bd):
    bn = 1000
    return pl.pallas_call(
        _dec_body,
        grid=(N // bn,),
        in_specs=[
            pl.BlockSpec((bn, AW2), lambda i: (i, 0)),
            pl.BlockSpec((bn, AW2), lambda i: (i, 0)),
            pl.BlockSpec((bn, AW2), lambda i: (i, 0)),
            pl.BlockSpec((bn, BW), lambda i: (i, 0)),
            pl.BlockSpec((8, 16), lambda i: (0, 0)),
            pl.BlockSpec((1, LAT), lambda i: (0, 0)),
            pl.BlockSpec((LAT, D), lambda i: (0, 0)),
            pl.BlockSpec((1, D), lambda i: (0, 0)),
        ],
        out_specs=pl.BlockSpec((bn, D), lambda i: (i, 0)),
        out_shape=jax.ShapeDtypeStruct((N, D), jnp.float32),
    )(acca, accb, A2, B2, cv2, b2v, Wd, bd)


def kernel(x, edge_index, W1, a_src1, a_dst1, b1, W2, a_src2, a_dst2, b2,
           Wd, bd):
    orig = _perm_orig()
    pmat = jnp.zeros((D, D), jnp.float32).at[orig, np.arange(D)].set(1.0)
    m16i = (np.arange(D)[:, None] // HID ==
            np.arange(16)[None, :] // 2).astype(np.float32)
    asrc_m = a_src1.reshape(D)[:, None] * m16i
    adst_m = a_dst1.reshape(D)[:, None] * m16i
    wbig = W1 @ jnp.concatenate([pmat, asrc_m, adst_m], axis=1)
    b1p = b1[orig].reshape(1, D)
    w2p = W2[orig, :]

    A1, B1, cv1 = _prep1(x, wbig)
    srcs = edge_index[0].reshape(NW, NCHUNK, K)
    dsts = edge_index[1].reshape(NW, NCHUNK, K)
    acc1 = _sc_edge1(A1, B1, srcs, dsts, cv1[0])
    A2, B2, cv2 = _prep2(acc1[0], acc1[1], A1, B1, cv1, b1p,
                         w2p, a_src2.reshape(1, LAT), a_dst2.reshape(1, LAT))
    acc2 = _sc_edge2(A2, B2, srcs, dsts, cv2[0])
    return _decode(acc2[0], acc2[1], A2, B2, cv2, b2.reshape(1, LAT),
                   Wd, bd.reshape(1, D))


# speedup vs baseline: 1.0357x; 1.0006x over previous
import functools

import jax
import jax.numpy as jnp
import numpy as np
from jax import lax
from jax.experimental import pallas as pl
from jax.experimental.pallas import tpu as pltpu
from jax.experimental.pallas import tpu_sc as plsc

N = 10000
E = 320000
D = 128
H = 8
HID = 16
LAT = 16

AW1 = D + 16
BW = 16
AW2 = 32

NC = 2
NS = 16
NW = NC * NS
EPW = E // NW
K = 100
NCHUNK = EPW // K
RPT = N // NS
ZR = 25

_NEG = 0.2


def _lrelu(v):
    return jnp.where(v > 0, v, v * _NEG)


def _perm_orig():
    newcol = np.arange(D)
    c = newcol // 16
    l = newcol % 16
    return (l // 2) * HID + 2 * c + l % 2


def _prep1_body(x_ref, w_ref, a1_ref, b1_ref, cv_ref):
    i = pl.program_id(0)
    nsteps = pl.num_programs(0)
    hb = jnp.dot(x_ref[...], w_ref[...], preferred_element_type=jnp.float32)
    a1_ref[...] = hb[:, :AW1]
    b1_ref[...] = hb[:, AW1:]
    cur = jnp.concatenate(
        [jnp.max(hb[:, D:AW1], axis=0, keepdims=True),
         jnp.max(hb[:, AW1:], axis=0, keepdims=True),
         jnp.zeros((6, 16), jnp.float32)], axis=0)

    @pl.when(i == 0)
    def _():
        cv_ref[...] = jnp.full((8, 16), -1e30, jnp.float32)

    mx = jnp.maximum(cv_ref[...], cur)
    cv_ref[...] = mx

    @pl.when(i == nsteps - 1)
    def _():
        c = _lrelu(mx[0:1, :] + mx[1:2, :])
        cv_ref[...] = jnp.concatenate([c, mx[1:8, :]], axis=0)


def _prep1(x, wbig):
    bn = 1000
    return pl.pallas_call(
        _prep1_body,
        grid=(N // bn,),
        in_specs=[
            pl.BlockSpec((bn, D), lambda i: (i, 0)),
            pl.BlockSpec((D, AW1 + BW), lambda i: (0, 0)),
        ],
        out_specs=[
            pl.BlockSpec((bn, AW1), lambda i: (i, 0)),
            pl.BlockSpec((bn, BW), lambda i: (i, 0)),
            pl.BlockSpec((8, 16), lambda i: (0, 0)),
        ],
        out_shape=[
            jax.ShapeDtypeStruct((N, AW1), jnp.float32),
            jax.ShapeDtypeStruct((N, BW), jnp.float32),
            jax.ShapeDtypeStruct((8, 16), jnp.float32),
        ],
    )(x, wbig)


def _make_sc_edge(aw, nheads, unroll, interleaved=False):
    mesh = plsc.VectorSubcoreMesh(core_axis_name="c", subcore_axis_name="s")

    @functools.partial(
        pl.kernel, mesh=mesh,
        compiler_params=pltpu.CompilerParams(use_tc_tiling_on_sc=False),
        out_type=jax.ShapeDtypeStruct((NC, N, aw), jnp.float32),
        scratch_types=[
            pltpu.VMEM_SHARED((N, aw), jnp.float32),
            [pltpu.VMEM((K,), jnp.int32)] * 4,
            [pltpu.VMEM((K,), jnp.int32)] * 4,
            [pltpu.VMEM((K, aw), jnp.float32)] * 2,
            [pltpu.VMEM((K, BW), jnp.float32)] * 2,
            pltpu.VMEM((16,), jnp.float32),
            pltpu.VMEM((ZR, aw), jnp.float32),
            [pltpu.SemaphoreType.DMA] * 4,
            [pltpu.SemaphoreType.DMA] * 2,
            [pltpu.SemaphoreType.DMA] * 2,
            [pltpu.SemaphoreType.DMA] * 2,
        ],
    )
    def sc_edge(a_hbm, b_hbm, src_hbm, dst_hbm, cv_hbm, out_hbm,
                acc_sh, idx_s, idx_d, rows_a, rows_b, cv_v, zero_v,
                ix, ga, gb, sc):
        c = lax.axis_index("c")
        s = lax.axis_index("s")
        w = c * NS + s

        zf = jnp.zeros((16,), jnp.float32)

        def zrow(rr, _):
            for jj in range(aw // 16):
                zero_v[rr, pl.ds(jj * 16, 16)] = zf
            return _

        lax.fori_loop(0, ZR, zrow, None)

        def zcp(k, _):
            pltpu.async_copy(zero_v, acc_sh.at[pl.ds(s * RPT + k * ZR, ZR)],
                             ga[0])
            return _

        lax.fori_loop(0, RPT // ZR, zcp, None)

        def zwait(k, _):
            pltpu.make_async_copy(
                zero_v, acc_sh.at[pl.ds(s * RPT + k * ZR, ZR)], ga[0]).wait()
            return _

        lax.fori_loop(0, RPT // ZR, zwait, None)

        pltpu.sync_copy(cv_hbm, cv_v)
        cvec = cv_v[...]
        plsc.subcore_barrier()

        def idx_fetch(tt, q):
            pltpu.async_copy(src_hbm.at[w, tt], idx_s[q], ix[q])
            pltpu.async_copy(dst_hbm.at[w, tt], idx_d[q], ix[q])

        def wait_idx(tt, q):
            pltpu.make_async_copy(src_hbm.at[w, tt], idx_s[q], ix[q]).wait()
            pltpu.make_async_copy(dst_hbm.at[w, tt], idx_d[q], ix[q]).wait()

        def gather(p, q):
            pltpu.async_copy(a_hbm.at[idx_s[q]], rows_a[p], ga[p])
            pltpu.async_copy(b_hbm.at[idx_d[q]], rows_b[p], gb[p])

        def wait_gather(p, q):
            pltpu.make_async_copy(a_hbm.at[idx_s[q]], rows_a[p], ga[p]).wait()
            pltpu.make_async_copy(b_hbm.at[idx_d[q]], rows_b[p], gb[p]).wait()

        def scatter(p, q):
            pltpu.async_copy(rows_a[p], acc_sh.at[idx_d[q]], sc[p], add=True)

        def wait_scatter(p, q):
            pltpu.make_async_copy(rows_a[p], acc_sh.at[idx_d[q]], sc[p]).wait()

        def compute(p):
            ra = rows_a[p]
            rb = rows_b[p]

            @plsc.parallel_loop(0, K, unroll=unroll)
            def edge(r):
                tail = ra[r, pl.ds(aw - 16, 16)]
                brow = rb[r, pl.ds(0, 16)]
                ee = jnp.exp(_lrelu(tail + brow) - cvec)
                if interleaved:
                    for j in range(nheads):
                        blk = ra[r, pl.ds(j * 16, 16)]
                        ra[r, pl.ds(j * 16, 16)] = blk * ee
                else:
                    for j in range(nheads):
                        bc = jnp.broadcast_to(ee[j], (16,))
                        blk = ra[r, pl.ds(j * 16, 16)]
                        ra[r, pl.ds(j * 16, 16)] = blk * bc
                ra[r, pl.ds(aw - 16, 16)] = ee

        idx_fetch(0, 0)
        idx_fetch(1, 1)
        idx_fetch(2, 2)
        wait_idx(0, 0)
        wait_idx(1, 1)
        gather(0, 0)

        def quad(u, _):
            t0 = u * 4
            for v in range(4):
                tt = t0 + v
                p = v % 2
                wait_gather(p, v)
                compute(p)

                if v == 0:
                    @pl.when(u > 0)
                    def _():
                        wait_scatter(1 - p, (v - 1) % 4)
                else:
                    wait_scatter(1 - p, (v - 1) % 4)

                @pl.when(tt + 2 < NCHUNK)
                def _():
                    wait_idx(tt + 2, (v + 2) % 4)

                @pl.when(tt + 1 < NCHUNK)
                def _():
                    gather(1 - p, (v + 1) % 4)

                @pl.when(tt + 3 < NCHUNK)
                def _():
                    idx_fetch(tt + 3, (v + 3) % 4)

                scatter(p, v)
            return _

        lax.fori_loop(0, NCHUNK // 4, quad, None)
        wait_scatter(1, (NCHUNK - 1) % 4)
        plsc.subcore_barrier()

        def drain(k, _):
            r0 = s * RPT + k * ZR
            pltpu.async_copy(acc_sh.at[pl.ds(r0, ZR)],
                             out_hbm.at[c, pl.ds(r0, ZR)], ga[0])
            return _

        lax.fori_loop(0, RPT // ZR, drain, None)

        def dwait(k, _):
            r0 = s * RPT + k * ZR
            pltpu.make_async_copy(acc_sh.at[pl.ds(r0, ZR)],
                                  out_hbm.at[c, pl.ds(r0, ZR)], ga[0]).wait()
            return _

        lax.fori_loop(0, RPT // ZR, dwait, None)

    return sc_edge


_sc_edge1 = _make_sc_edge(AW1, H, 20, interleaved=True)
_sc_edge2 = _make_sc_edge(AW2, 1, 16)


def _prep2_body(acca_ref, accb_ref, a1_ref, b1_ref, cv1_ref, b1v_ref,
                w2_ref, asf2_ref, adf2_ref, a2_ref, b2_ref, cv2_ref):
    i = pl.program_id(0)
    nsteps = pl.num_programs(0)
    acc = acca_ref[...] + accb_ref[...]
    msg_p = acc[:, :D]
    den16 = acc[:, D:AW1]
    h_p = a1_ref[:, :D]
    asrc16 = a1_ref[:, D:AW1]
    adst16 = b1_ref[...]
    c1 = cv1_ref[0:1, :]
    ee_s = jnp.exp(_lrelu(asrc16 + adst16) - c1)
    den16 = den16 + ee_s
    row = lax.broadcasted_iota(jnp.int32, (16, D), 0)
    colm = lax.broadcasted_iota(jnp.int32, (16, D), 1) % 16
    tmat = (row == colm).astype(jnp.float32)
    ee128 = jnp.dot(ee_s, tmat, preferred_element_type=jnp.float32)
    den_p = jnp.dot(den16, tmat, preferred_element_type=jnp.float32)
    h1p = jnp.maximum(
        (msg_p + ee128 * h_p) / (den_p + 1e-16) + b1v_ref[...], 0.0)
    h2 = jnp.dot(h1p, w2_ref[...], preferred_element_type=jnp.float32)
    lane = lax.broadcasted_iota(jnp.int32, h2.shape, 1)
    asum = jnp.sum(h2 * asf2_ref[...], axis=1, keepdims=True)
    adum = jnp.sum(h2 * adf2_ref[...], axis=1, keepdims=True)
    asrc2 = jnp.where(lane == 0, asum, 0.0)
    adst2 = jnp.where(lane == 0, adum, 0.0)
    a2_ref[...] = jnp.concatenate([h2, asrc2], axis=1)
    b2_ref[...] = adst2
    cur = jnp.concatenate(
        [jnp.max(asrc2, axis=0, keepdims=True),
         jnp.max(adst2, axis=0, keepdims=True),
         jnp.zeros((6, 16), jnp.float32)], axis=0)

    @pl.when(i == 0)
    def _():
        cv2_ref[...] = jnp.full((8, 16), -1e30, jnp.float32)

    mx = jnp.maximum(cv2_ref[...], cur)
    cv2_ref[...] = mx

    @pl.when(i == nsteps - 1)
    def _():
        c2 = _lrelu(mx[0:1, :] + mx[1:2, :])
        cv2_ref[...] = jnp.concatenate([c2, mx[1:8, :]], axis=0)


def _prep2(acca, accb, A1, B1, cv1, b1v, W2, asf2, adf2):
    bn = 1000
    return pl.pallas_call(
        _prep2_body,
        grid=(N // bn,),
        in_specs=[
            pl.BlockSpec((bn, AW1), lambda i: (i, 0)),
            pl.BlockSpec((bn, AW1), lambda i: (i, 0)),
            pl.BlockSpec((bn, AW1), lambda i: (i, 0)),
            pl.BlockSpec((bn, BW), lambda i: (i, 0)),
            pl.BlockSpec((8, 16), lambda i: (0, 0)),
            pl.BlockSpec((1, D), lambda i: (0, 0)),
            pl.BlockSpec((D, LAT), lambda i: (0, 0)),
            pl.BlockSpec((1, LAT), lambda i: (0, 0)),
            pl.BlockSpec((1, LAT), lambda i: (0, 0)),
        ],
        out_specs=[
            pl.BlockSpec((bn, AW2), lambda i: (i, 0)),
            pl.BlockSpec((bn, BW), lambda i: (i, 0)),
            pl.BlockSpec((8, 16), lambda i: (0, 0)),
        ],
        out_shape=[
            jax.ShapeDtypeStruct((N, AW2), jnp.float32),
            jax.ShapeDtypeStruct((N, BW), jnp.float32),
            jax.ShapeDtypeStruct((8, 16), jnp.float32),
        ],
    )(acca, accb, A1, B1, cv1, b1v, W2, asf2, adf2)


def _dec_body(acca_ref, accb_ref, a2_ref, b2_ref, cv2_ref, b2v_ref,
              wd_ref, bd_ref, o_ref):
    acc = acca_ref[...] + accb_ref[...]
    msg = acc[:, :LAT]
    den = acc[:, LAT:AW2]
    h2 = a2_ref[:, :LAT]
    asrc2 = a2_ref[:, LAT:AW2]
    adst2 = b2_ref[...]
    c2 = cv2_ref[0:1, :]
    ee_s = jnp.exp(_lrelu(asrc2 + adst2) - c2)
    den = den + ee_s
    lane = lax.broadcasted_iota(jnp.int32, msg.shape, 1)
    ee0 = jnp.sum(jnp.where(lane == 0, ee_s, 0.0), axis=1, keepdims=True)
    den0 = jnp.sum(jnp.where(lane == 0, den, 0.0), axis=1, keepdims=True)
    z = (msg + ee0 * h2) / (den0 + 1e-16) + b2v_ref[...]
    o_ref[...] = jnp.dot(z, wd_ref[...], preferred_element_type=jnp.float32) \
        + bd_ref[...]


def _decode(acca, accb, A2, B2, cv2, b2v, Wd, bd):
    bn = 1000
    return pl.pallas_call(
        _dec_body,
        grid=(N // bn,),
        in_specs=[
            pl.BlockSpec((bn, AW2), lambda i: (i, 0)),
            pl.BlockSpec((bn, AW2), lambda i: (i, 0)),
            pl.BlockSpec((bn, AW2), lambda i: (i, 0)),
            pl.BlockSpec((bn, BW), lambda i: (i, 0)),
            pl.BlockSpec((8, 16), lambda i: (0, 0)),
            pl.BlockSpec((1, LAT), lambda i: (0, 0)),
            pl.BlockSpec((LAT, D), lambda i: (0, 0)),
            pl.BlockSpec((1, D), lambda i: (0, 0)),
        ],
        out_specs=pl.BlockSpec((bn, D), lambda i: (i, 0)),
        out_shape=jax.ShapeDtypeStruct((N, D), jnp.float32),
    )(acca, accb, A2, B2, cv2, b2v, Wd, bd)


def kernel(x, edge_index, W1, a_src1, a_dst1, b1, W2, a_src2, a_dst2, b2,
           Wd, bd):
    orig = _perm_orig()
    pmat = jnp.zeros((D, D), jnp.float32).at[orig, np.arange(D)].set(1.0)
    m16i = (np.arange(D)[:, None] // HID ==
            np.arange(16)[None, :] // 2).astype(np.float32)
    asrc_m = a_src1.reshape(D)[:, None] * m16i
    adst_m = a_dst1.reshape(D)[:, None] * m16i
    wbig = W1 @ jnp.concatenate([pmat, asrc_m, adst_m], axis=1)
    b1p = b1[orig].reshape(1, D)
    w2p = W2[orig, :]

    A1, B1, cv1 = _prep1(x, wbig)
    srcs = edge_index[0].reshape(NW, NCHUNK, K)
    dsts = edge_index[1].reshape(NW, NCHUNK, K)
    acc1 = _sc_edge1(A1, B1, srcs, dsts, cv1[0])
    A2, B2, cv2 = _prep2(acc1[0], acc1[1], A1, B1, cv1, b1p,
                         w2p, a_src2.reshape(1, LAT), a_dst2.reshape(1, LAT))
    acc2 = _sc_edge2(A2, B2, srcs, dsts, cv2[0])
    return _decode(acc2[0], acc2[1], A2, B2, cv2, b2.reshape(1, LAT),
                   Wd, bd.reshape(1, D))
